# Initial kernel scaffold; baseline (speedup 1.0000x reference)
import jax
import jax.numpy as jnp
from jax.experimental import pallas as pl


def kernel(x2d, idx):
    raise NotImplementedError("write your pallas kernel here")


# speedup vs baseline: 17.8881x; 17.8881x over previous
import functools

import jax
import jax.numpy as jnp
from jax import lax
from jax.experimental import pallas as pl
from jax.experimental.pallas import tpu as pltpu
from jax.experimental.pallas import tpu_sc as plsc

W3, H3, D3 = 60, 36, 60
V = W3 * H3 * D3
BS, C, N = 4, 64, 19200
HALF = V // 2
NVEC = N // 16
CG = 16


def _body(x_hbm, i_hbm, out_hbm, vrel_v, src_v, slab_v):
    wid = lax.axis_index("s") * 2 + lax.axis_index("c")
    b = wid // 8
    half = (wid // 4) % 2
    cg = wid % 4
    c0 = cg * CG
    lo = pl.multiple_of(half * HALF, 8)

    pltpu.sync_copy(i_hbm.at[pl.ds(pl.multiple_of(b * N, 16), N)], vrel_v)

    def prep(j, carry):
        v = vrel_v[pl.ds(j * 16, 16)]
        t = ((v >> 2) * 34953) >> 19
        d = v - t * 60
        w = ((t >> 2) * 7282) >> 16
        h = t - w * 36
        vrel_v[pl.ds(j * 16, 16)] = d * 2160 + h * 60 + w - lo
        return carry

    lax.fori_loop(0, NVEC, prep, 0)

    zeros16 = jnp.zeros((16,), jnp.float32)
    false16 = jnp.zeros((16,), jnp.bool_)
    G = 60
    NG = NVEC // G

    def per_channel(ch, carry):
        c = c0 + ch
        src_off = pl.multiple_of((b * C + c) * N, 16)
        pltpu.sync_copy(x_hbm.at[pl.ds(src_off, N)], src_v)

        def zero(i, cz):
            base = i * 80
            slab_v[pl.ds(base, 16)] = zeros16
            slab_v[pl.ds(base + 16, 16)] = zeros16
            slab_v[pl.ds(base + 32, 16)] = zeros16
            slab_v[pl.ds(base + 48, 16)] = zeros16
            slab_v[pl.ds(base + 64, 16)] = zeros16
            return cz

        lax.fori_loop(0, HALF // 80, zero, 0)

        def group(g, cg_):
            def mrg(j, dacc):
                kk = vrel_v[pl.ds(j * 16, 16)]
                val = src_v[pl.ds(j * 16, 16)]
                m0 = (kk >= 0) & (kk < HALF)
                cur = plsc.load_gather(slab_v, [kk], mask=m0)
                plsc.store_scatter(slab_v, [kk], jnp.maximum(cur, val), mask=m0)
                back = plsc.load_gather(slab_v, [kk], mask=m0)
                return dacc | (m0 & (back < val))

            dirty = lax.fori_loop(g * G, (g + 1) * G, mrg, false16)

            @pl.when(jnp.any(dirty))
            def _fixgroup():
                def fix(j, cf):
                    kk = vrel_v[pl.ds(j * 16, 16)]
                    val = src_v[pl.ds(j * 16, 16)]
                    m0 = (kk >= 0) & (kk < HALF)
                    back = plsc.load_gather(slab_v, [kk], mask=m0)
                    m1 = m0 & (back < val)

                    @pl.when(jnp.any(m1))
                    def _rounds():
                        def rnd(r, m):
                            c2 = plsc.load_gather(slab_v, [kk], mask=m)
                            plsc.store_scatter(
                                slab_v, [kk], jnp.maximum(c2, val), mask=m
                            )
                            b2 = plsc.load_gather(slab_v, [kk], mask=m)
                            return m & (b2 < val)

                        lax.fori_loop(0, 16, rnd, m1)

                    return cf

                lax.fori_loop(g * G, (g + 1) * G, fix, 0)

            return cg_

        lax.fori_loop(0, NG, group, 0)
        out_off = pl.multiple_of((b * C + c) * V + half * HALF, 16)
        pltpu.sync_copy(slab_v, out_hbm.at[pl.ds(out_off, HALF)])
        return carry

    lax.fori_loop(0, CG, per_channel, 0)


@jax.jit
def _scatter3d(x, ind):
    mesh = plsc.VectorSubcoreMesh(
        core_axis_name="c", subcore_axis_name="s", num_cores=2, num_subcores=16
    )
    return pl.kernel(
        _body,
        out_type=jax.ShapeDtypeStruct((BS * C * V,), jnp.float32),
        mesh=mesh,
        compiler_params=pltpu.CompilerParams(needs_layout_passes=False),
        scratch_types=[
            pltpu.VMEM((N,), jnp.int32),
            pltpu.VMEM((N,), jnp.float32),
            pltpu.VMEM((HALF,), jnp.float32),
        ],
    )(x, ind)


def kernel(x2d, idx):
    x = x2d.reshape(BS * C * N)
    ind = idx.reshape(BS * N).astype(jnp.int32)
    out = _scatter3d(x, ind)
    return out.reshape(BS, C, D3, H3, W3)


# speedup vs baseline: 17.9649x; 1.0043x over previous
import jax
import jax.numpy as jnp
from jax import lax
from jax.experimental import pallas as pl
from jax.experimental.pallas import tpu as pltpu
from jax.experimental.pallas import tpu_sc as plsc

W3, H3, D3 = 60, 36, 60
V = W3 * H3 * D3
BS, C, N = 4, 64, 19200
QUART = V // 4
NVEC = N // 16
CG = 32
LCAP = N + 16
G = 60
NG = NVEC // G


def _body(x_hbm, i_hbm, out_hbm, idx_v, list_v, src_v, slab_a, slab_b,
          sem_a, sem_b):
    wid = lax.axis_index("s") * 2 + lax.axis_index("c")
    b = wid // 8
    quarter = (wid // 2) % 4
    c0 = (wid % 2) * CG
    lo = quarter * QUART
    iota16 = lax.iota(jnp.int32, 16)
    zeros16 = jnp.zeros((16,), jnp.float32)
    false16 = jnp.zeros((16,), jnp.bool_)

    pltpu.sync_copy(i_hbm.at[pl.ds(pl.multiple_of(b * N, 16), N)], idx_v)

    def prep(j, off):
        v = idx_v[pl.ds(j * 16, 16)]
        t = ((v >> 2) * 34953) >> 19
        d = v - t * 60
        w = ((t >> 2) * 7282) >> 16
        h = t - w * 36
        vp = d * 2160 + h * 60 + w - lo
        m = (vp >= 0) & (vp < QUART)
        packed = vp * 32768 + (iota16 + j * 16)
        rank = plsc.cumsum(jnp.where(m, 1, 0))
        plsc.store_scatter(list_v, [off + rank - 1], packed, mask=m)
        return off + plsc.all_reduce_population_count(m)

    off = lax.fori_loop(0, NVEC, prep, jnp.zeros((16,), jnp.int32))
    cnt_s = off[0]

    def zero(i, cz):
        base = i * 80
        for k in range(0, 80, 16):
            slab_a[pl.ds(base + k, 16)] = zeros16
            slab_b[pl.ds(base + k, 16)] = zeros16
        return cz

    lax.fori_loop(0, QUART // 80, zero, 0)

    def run_channel(c, slab_v, first):
        src_off = pl.multiple_of((b * C + c) * N, 16)
        pltpu.sync_copy(x_hbm.at[pl.ds(src_off, N)], src_v)

        @pl.when(jnp.logical_not(first))
        def _clear():
            def clr(j, cc):
                p = list_v[pl.ds(j * 16, 16)]
                tail = (iota16 + j * 16) < off
                plsc.store_scatter(slab_v, [p >> 15], zeros16, mask=tail)
                return cc

            def clr_group(g, cc):
                @pl.when(g * (G * 16) < cnt_s)
                def _g():
                    lax.fori_loop(g * G, (g + 1) * G, clr, 0)
                return cc

            lax.fori_loop(0, NG, clr_group, 0)

        def mrg(j, dacc):
            p = list_v[pl.ds(j * 16, 16)]
            kk = p >> 15
            n = p & 32767
            tail = (iota16 + j * 16) < off
            val = plsc.load_gather(src_v, [n], mask=tail)
            cur = plsc.load_gather(slab_v, [kk], mask=tail)
            plsc.store_scatter(slab_v, [kk], jnp.maximum(cur, val), mask=tail)
            back = plsc.load_gather(slab_v, [kk], mask=tail)
            return dacc | (tail & (back < val))

        def fix(j, cf):
            p = list_v[pl.ds(j * 16, 16)]
            kk = p >> 15
            n = p & 32767
            tail = (iota16 + j * 16) < off
            val = plsc.load_gather(src_v, [n], mask=tail)
            back = plsc.load_gather(slab_v, [kk], mask=tail)
            m1 = tail & (back < val)

            @pl.when(jnp.any(m1))
            def _rounds():
                def rnd(r, m):
                    c2 = plsc.load_gather(slab_v, [kk], mask=m)
                    plsc.store_scatter(slab_v, [kk], jnp.maximum(c2, val), mask=m)
                    b2 = plsc.load_gather(slab_v, [kk], mask=m)
                    return m & (b2 < val)

                lax.fori_loop(0, 16, rnd, m1)

            return cf

        def group(g, cg_):
            @pl.when(g * (G * 16) < cnt_s)
            def _g():
                dirty = lax.fori_loop(g * G, (g + 1) * G, mrg, false16)

                @pl.when(jnp.any(dirty))
                def _fixgroup():
                    lax.fori_loop(g * G, (g + 1) * G, fix, 0)

            return cg_

        lax.fori_loop(0, NG, group, 0)

    def out_ref_for(c):
        out_off = pl.multiple_of((b * C + c) * V + lo, 16)
        return out_hbm.at[pl.ds(out_off, QUART)]

    def pair(i, carry):
        c_even = c0 + 2 * i
        c_odd = c_even + 1

        @pl.when(i > 0)
        def _wa():
            pltpu.make_async_copy(slab_a, out_ref_for(c_even), sem_a).wait()

        run_channel(c_even, slab_a, i == 0)
        pltpu.async_copy(slab_a, out_ref_for(c_even), sem_a)

        @pl.when(i > 0)
        def _wb():
            pltpu.make_async_copy(slab_b, out_ref_for(c_odd), sem_b).wait()

        run_channel(c_odd, slab_b, i == 0)
        pltpu.async_copy(slab_b, out_ref_for(c_odd), sem_b)
        return carry

    lax.fori_loop(0, CG // 2, pair, 0)
    pltpu.make_async_copy(slab_a, out_ref_for(c0), sem_a).wait()
    pltpu.make_async_copy(slab_b, out_ref_for(c0), sem_b).wait()


@jax.jit
def _scatter3d(x, ind):
    mesh = plsc.VectorSubcoreMesh(
        core_axis_name="c", subcore_axis_name="s", num_cores=2, num_subcores=16
    )
    return pl.kernel(
        _body,
        out_type=jax.ShapeDtypeStruct((BS * C * V,), jnp.float32),
        mesh=mesh,
        compiler_params=pltpu.CompilerParams(needs_layout_passes=False),
        scratch_types=[
            pltpu.VMEM((N,), jnp.int32),
            pltpu.VMEM((LCAP,), jnp.int32),
            pltpu.VMEM((N,), jnp.float32),
            pltpu.VMEM((QUART,), jnp.float32),
            pltpu.VMEM((QUART,), jnp.float32),
            pltpu.SemaphoreType.DMA,
            pltpu.SemaphoreType.DMA,
        ],
    )(x, ind)


def kernel(x2d, idx):
    x = x2d.reshape(BS * C * N)
    ind = idx.reshape(BS * N).astype(jnp.int32)
    out = _scatter3d(x, ind)
    return out.reshape(BS, C, D3, H3, W3)
